# baseline (speedup 1.0000x reference)
import functools

import jax
import jax.numpy as jnp
import numpy as np
from jax.experimental import pallas as pl
from jax.experimental.pallas import tpu as pltpu
from jax.experimental.shard_map import shard_map
from jax.sharding import Mesh, PartitionSpec as P

_N = 10000
_F = 128
_H = 128
_C = 10
_K = 8
_ALPHA = 0.1

_F8 = jnp.float8_e4m3fn


def _encoder_kernel(x_ref, w1_ref, b1_ref, w2_ref, b2_ref, z_ref):
    h = jax.lax.dot_general(
        x_ref[...], w1_ref[...], (((1,), (1,)), ((), ())),
        preferred_element_type=jnp.float32)
    h = jax.nn.relu(h + b1_ref[...])
    z = jax.lax.dot_general(
        h, w2_ref[...], (((1,), (1,)), ((), ())),
        preferred_element_type=jnp.float32)
    z_ref[...] = z + b2_ref[...]


def _encoder(x, W1, b1, W2, b2):
    m = x.shape[0]
    bm = 1000
    return pl.pallas_call(
        _encoder_kernel,
        grid=(m // bm,),
        in_specs=[
            pl.BlockSpec((bm, _F), lambda i: (i, 0)),
            pl.BlockSpec((_H, _F), lambda i: (0, 0)),
            pl.BlockSpec((1, _H), lambda i: (0, 0)),
            pl.BlockSpec((_C, _H), lambda i: (0, 0)),
            pl.BlockSpec((1, _C), lambda i: (0, 0)),
        ],
        out_specs=pl.BlockSpec((bm, _C), lambda i: (i, 0)),
        out_shape=jax.ShapeDtypeStruct((m, _C), jnp.float32),
    )(x, W1, b1.reshape(1, _H), W2, b2.reshape(1, _C))


def _quant_step0_kernel(bmq, adj_ref, z_ref, adj8_ref, cur1_ref):
    i = pl.program_id(0)
    a8 = adj_ref[...].astype(_F8)
    adj8_ref[...] = a8
    zb = z_ref[...].astype(jnp.bfloat16)
    y = jax.lax.dot_general(
        a8.astype(jnp.bfloat16), zb, (((1,), (0,)), ((), ())),
        preferred_element_type=jnp.float32)
    y = y * (1.0 - _ALPHA)
    y = y + _ALPHA * z_ref[pl.ds(i * bmq, bmq), :]
    cur1_ref[...] = y.astype(jnp.bfloat16)


def _quant_step0(adj_rows, z_full):
    m = adj_rows.shape[0]
    bmq = 200
    return pl.pallas_call(
        functools.partial(_quant_step0_kernel, bmq),
        grid=(m // bmq,),
        in_specs=[
            pl.BlockSpec((bmq, _N), lambda i: (i, 0)),
            pl.BlockSpec((_N, _C), lambda i: (0, 0)),
        ],
        out_specs=[
            pl.BlockSpec((bmq, _N), lambda i: (i, 0)),
            pl.BlockSpec((bmq, _C), lambda i: (i, 0)),
        ],
        out_shape=[
            jax.ShapeDtypeStruct((m, _N), _F8),
            jax.ShapeDtypeStruct((m, _C), jnp.bfloat16),
        ],
    )(adj_rows, z_full)


def _step_kernel(last, adj8_ref, cur_ref, z_ref, out_ref):
    y = jax.lax.dot_general(
        adj8_ref[...].astype(jnp.bfloat16), cur_ref[...],
        (((1,), (0,)), ((), ())),
        preferred_element_type=jnp.float32)
    y = y * (1.0 - _ALPHA)
    y = y + _ALPHA * z_ref[...]
    if not last:
        out_ref[...] = y.astype(jnp.bfloat16)
    else:
        m = jnp.max(y, axis=1, keepdims=True)
        shifted = y - m
        lse = jnp.log(jnp.sum(jnp.exp(shifted), axis=1, keepdims=True))
        out_ref[...] = shifted - lse


def _step(adj8_rows, cur_full_bf16, z_rows, last):
    m = adj8_rows.shape[0]
    bm = 1000
    odtype = jnp.float32 if last else jnp.bfloat16
    return pl.pallas_call(
        functools.partial(_step_kernel, last),
        grid=(m // bm,),
        in_specs=[
            pl.BlockSpec((bm, _N), lambda i: (i, 0)),
            pl.BlockSpec((_N, _C), lambda i: (0, 0)),
            pl.BlockSpec((bm, _C), lambda i: (i, 0)),
        ],
        out_specs=pl.BlockSpec((bm, _C), lambda i: (i, 0)),
        out_shape=jax.ShapeDtypeStruct((m, _C), odtype),
    )(adj8_rows, cur_full_bf16, z_rows)


def _run_local(x_rows, adj_rows, W1, b1, W2, b2, gather):
    z_rows = _encoder(x_rows, W1, b1, W2, b2)
    z_full = gather(z_rows)
    adj8_rows, cur1_rows = _quant_step0(adj_rows, z_full)
    cur = gather(cur1_rows)
    for t in range(1, _K):
        res = _step(adj8_rows, cur, z_rows, t == _K - 1)
        if t < _K - 1:
            cur = gather(res)
    return res


def kernel(x, adj, W1, b1, W2, b2):
    devs = jax.devices()
    if len(devs) >= 2:
        mesh = Mesh(np.array(devs[:2]), ("d",))

        def gather(a):
            return jax.lax.all_gather(a, "d", axis=0, tiled=True)

        run = shard_map(
            functools.partial(_run_local, gather=gather),
            mesh=mesh,
            in_specs=(P("d"), P("d"), P(), P(), P(), P()),
            out_specs=P("d"),
            check_rep=False,
        )
        return run(x, adj, W1, b1, W2, b2)
    return _run_local(x, adj, W1, b1, W2, b2, gather=lambda a: a)


# speedup vs baseline: 1.8299x; 1.8299x over previous
import jax
import jax.numpy as jnp
from jax.experimental import pallas as pl
from jax.experimental.pallas import tpu as pltpu

_N = 10000
_F = 128
_H = 128
_C = 10
_K = 8
_ALPHA = 0.1

_BMQ = 400
_BM = 400


def _encoder_kernel(x_ref, w1_ref, b1_ref, w2_ref, b2_ref, z_ref):
    h = jax.lax.dot_general(
        x_ref[...], w1_ref[...], (((1,), (1,)), ((), ())),
        preferred_element_type=jnp.float32)
    h = jax.nn.relu(h + b1_ref[...])
    z = jax.lax.dot_general(
        h, w2_ref[...], (((1,), (1,)), ((), ())),
        preferred_element_type=jnp.float32)
    z_ref[...] = z + b2_ref[...]


def _quant_step0_kernel(adj_ref, z_ref, adj8_ref, cur1_ref):
    i = pl.program_id(0)
    a8 = adj_ref[...].astype(jnp.float8_e4m3fn)
    adj8_ref[...] = a8
    zb = z_ref[...].astype(jnp.bfloat16)
    y = jax.lax.dot_general(
        a8.astype(jnp.bfloat16), zb, (((1,), (0,)), ((), ())),
        preferred_element_type=jnp.float32)
    y = y * (1.0 - _ALPHA)
    y = y + _ALPHA * z_ref[pl.ds(i * _BMQ, _BMQ), :]
    cur1_ref[...] = y.astype(jnp.bfloat16)


def _prop_kernel(adj8_ref, z_ref, cur1_ref, out_ref, cur_ref):
    k = pl.program_id(0)
    i = pl.program_id(1)

    @pl.when(jnp.logical_and(k == 0, i == 0))
    def _():
        cur_ref[0] = cur1_ref[...]

    prev = cur_ref[jnp.remainder(k, 2)]
    y = jax.lax.dot_general(
        adj8_ref[...].astype(jnp.bfloat16), prev,
        (((1,), (0,)), ((), ())),
        preferred_element_type=jnp.float32)
    y = y * (1.0 - _ALPHA)
    y = y + _ALPHA * z_ref[...]
    cur_ref[jnp.remainder(k + 1, 2), pl.ds(i * _BM, _BM), :] = (
        y.astype(jnp.bfloat16))

    @pl.when(k == _K - 2)
    def _():
        m = jnp.max(y, axis=1, keepdims=True)
        shifted = y - m
        lse = jnp.log(jnp.sum(jnp.exp(shifted), axis=1, keepdims=True))
        out_ref[pl.ds(i * _BM, _BM), :] = shifted - lse


def kernel(x, adj, W1, b1, W2, b2):
    z = pl.pallas_call(
        _encoder_kernel,
        grid=(_N // 1000,),
        in_specs=[
            pl.BlockSpec((1000, _F), lambda i: (i, 0)),
            pl.BlockSpec((_H, _F), lambda i: (0, 0)),
            pl.BlockSpec((1, _H), lambda i: (0, 0)),
            pl.BlockSpec((_C, _H), lambda i: (0, 0)),
            pl.BlockSpec((1, _C), lambda i: (0, 0)),
        ],
        out_specs=pl.BlockSpec((1000, _C), lambda i: (i, 0)),
        out_shape=jax.ShapeDtypeStruct((_N, _C), jnp.float32),
    )(x, W1, b1.reshape(1, _H), W2, b2.reshape(1, _C))

    adj8, cur1 = pl.pallas_call(
        _quant_step0_kernel,
        grid=(_N // _BMQ,),
        in_specs=[
            pl.BlockSpec((_BMQ, _N), lambda i: (i, 0)),
            pl.BlockSpec((_N, _C), lambda i: (0, 0)),
        ],
        out_specs=[
            pl.BlockSpec((_BMQ, _N), lambda i: (i, 0)),
            pl.BlockSpec((_BMQ, _C), lambda i: (i, 0)),
        ],
        out_shape=[
            jax.ShapeDtypeStruct((_N, _N), jnp.float8_e4m3fn),
            jax.ShapeDtypeStruct((_N, _C), jnp.bfloat16),
        ],
    )(adj, z)

    out = pl.pallas_call(
        _prop_kernel,
        grid=(_K - 1, _N // _BM),
        in_specs=[
            pl.BlockSpec((_BM, _N), lambda k, i: (i, 0)),
            pl.BlockSpec((_BM, _C), lambda k, i: (i, 0)),
            pl.BlockSpec((_N, _C), lambda k, i: (0, 0)),
        ],
        out_specs=pl.BlockSpec((_N, _C), lambda k, i: (0, 0)),
        out_shape=jax.ShapeDtypeStruct((_N, _C), jnp.float32),
        scratch_shapes=[pltpu.VMEM((2, _N, _C), jnp.bfloat16)],
    )(adj8, z, cur1)
    return out
